# baseline (speedup 1.0000x reference)
import functools

import jax
import jax.numpy as jnp
from jax import lax
from jax.experimental import pallas as pl
from jax.experimental.pallas import tpu as pltpu
from jax.experimental.pallas import tpu_sc as plsc

L = 16
NC = 2
NS = 16
NW = NC * NS
CHR = 16
CH = CHR * 128
EPS = 1e-5

_SC_PARAMS = pltpu.CompilerParams(
    use_tc_tiling_on_sc=False, needs_layout_passes=False)


@functools.cache
def _sc_mesh():
    return plsc.VectorSubcoreMesh(
        core_axis_name="c", subcore_axis_name="s",
        num_cores=NC, num_subcores=NS)


def _al(v, m):
    return pl.multiple_of(v, m)


def _tile_chunks(rows, chunk_rows):
    nchunk = rows // chunk_rows
    tail = rows % chunk_rows
    cpw = pl.cdiv(nchunk, NW)
    return nchunk, tail, cpw


def _run_blocks(rows, chunk_rows, do_block, carry):
    wid = lax.axis_index("s") * NC + lax.axis_index("c")
    nchunk, tail, cpw = _tile_chunks(rows, chunk_rows)
    c0 = jnp.minimum(wid * cpw, nchunk)
    c1 = jnp.minimum(c0 + cpw, nchunk)

    def chunk(ch, c):
        return do_block((c0 + ch) * chunk_rows, chunk_rows, c)

    carry = lax.fori_loop(0, c1 - c0, chunk, carry)
    if tail:
        carry = lax.cond(
            wid == NW - 1,
            lambda c: do_block(jnp.int32(nchunk * chunk_rows), tail, c),
            lambda c: c, carry)
    return wid, carry


def _make_sc_pass1(e_real, use_ea, relu):
    rows = e_real // 128
    chr1 = 8
    nec = chr1 * 128
    nchunk = rows // chr1
    tail = rows % chr1
    cpw = pl.cdiv(nchunk, NW)

    def body(xs_hbm, xn_hbm, dst2_hbm, src2_hbm, ea0_hbm, ea1_hbm, w_hbm,
             m_hbm, stats_hbm,
             idxd_v, idxs_v, ea0_v, ea1_v, rowd_v, rows_v, mb_v, w_v, st_v,
             semi0, semi1, semg0, semg1, semw):
        semi = (semi0, semi1)
        semg = (semg0, semg1)
        wid = lax.axis_index("s") * NC + lax.axis_index("c")
        c0 = jnp.minimum(wid * cpw, nchunk)
        c1 = jnp.minimum(c0 + cpw, nchunk)
        nch = c1 - c0
        pltpu.sync_copy(w_hbm, w_v)
        we0 = w_v[0]
        we1 = w_v[1]
        zero = jnp.zeros((L,), jnp.float32)

        def row_of(ch):
            return _al(jnp.minimum(c0 + ch, c1 - 1) * chr1, 8)

        def load_descs(ch, b, issue):
            row = row_of(ch)
            f = pltpu.async_copy if issue else (
                lambda s, d, m: pltpu.make_async_copy(s, d, m))
            ds = [f(dst2_hbm.at[pl.ds(row, chr1)], idxd_v.at[b], semi[b]),
                  f(src2_hbm.at[pl.ds(row, chr1)], idxs_v.at[b], semi[b])]
            if use_ea:
                ds.append(f(ea0_hbm.at[pl.ds(row, chr1)], ea0_v.at[b],
                            semi[b]))
                ds.append(f(ea1_hbm.at[pl.ds(row, chr1)], ea1_v.at[b],
                            semi[b]))
            return ds

        def gather_descs(b, issue):
            f = pltpu.async_copy if issue else (
                lambda s, d, m: pltpu.make_async_copy(s, d, m))
            ds = []
            for j in range(chr1):
                ds.append(f(xs_hbm.at[idxd_v.at[b, j]],
                            rowd_v.at[b, pl.ds(j * 128, 128)], semg[b]))
                ds.append(f(xn_hbm.at[idxs_v.at[b, j]],
                            rows_v.at[b, pl.ds(j * 128, 128)], semg[b]))
            return ds

        def mw_desc(ch, b, issue):
            base = _al(row_of(ch) * 128, 1024)
            src = mb_v.at[b]
            dst = m_hbm.at[pl.ds(base, nec)]
            if issue:
                return pltpu.async_copy(src, dst, semw)
            return pltpu.make_async_copy(src, dst, semw)

        def compute(b, carry):
            def grp16(g, c2):
                a1, a2, b1, b2 = c2
                if use_ea:
                    va0 = ea0_v[b, g // 8, pl.ds((g % 8) * L, L)]
                    va1 = ea1_v[b, g // 8, pl.ds((g % 8) * L, L)]
                for u in range(L):
                    e = g * L + u
                    m = rowd_v[b, e] + rows_v[b, e]
                    if use_ea:
                        m = m + va0[u] * we0 + va1[u] * we1
                    if relu:
                        m = jnp.maximum(m, 0.0)
                    mb_v[b, e] = m
                    if u % 2 == 0:
                        a1 = a1 + m
                        a2 = a2 + m * m
                    else:
                        b1 = b1 + m
                        b2 = b2 + m * m
                return (a1, a2, b1, b2)

            return lax.fori_loop(0, nec // L, grp16, carry)

        def process(ch, b, carry):
            @pl.when(ch + 1 < nch)
            def _():
                for d in load_descs(ch + 1, 1 - b, False):
                    d.wait()
                gather_descs(1 - b, True)

            for d in gather_descs(b, False):
                d.wait()

            @pl.when(ch >= 2)
            def _():
                mw_desc(ch - 2, b, False).wait()

            carry = compute(b, carry)

            @pl.when(ch + 2 < nch)
            def _():
                load_descs(ch + 2, b, True)

            mw_desc(ch, b, True)
            return carry

        for d in load_descs(0, 0, True):
            d.wait()
        gather_descs(0, True)

        @pl.when(nch > 1)
        def _():
            load_descs(1, 1, True)

        def pair(g, carry):
            carry = process(2 * g, 0, carry)
            return process(2 * g + 1, 1, carry)

        carry = lax.fori_loop(0, nch // 2, pair, (zero, zero, zero, zero))
        carry = lax.cond(
            nch % 2 == 1,
            lambda c: process(nch - 1, 0, c),
            lambda c: c, carry)
        @pl.when(nch >= 2)
        def _():
            mw_desc(nch - 2, (nch - 2) % 2, False).wait()

        mw_desc(nch - 1, (nch - 1) % 2, False).wait()

        def do_tail(carry):
            row = _al(jnp.int32(nchunk * chr1), 8)
            nr = tail
            ne = nr * 128
            base = _al(row * 128, 1024)
            pltpu.sync_copy(dst2_hbm.at[pl.ds(row, nr)],
                            idxd_v.at[0, pl.ds(0, nr)])
            pltpu.sync_copy(src2_hbm.at[pl.ds(row, nr)],
                            idxs_v.at[0, pl.ds(0, nr)])
            if use_ea:
                pltpu.sync_copy(ea0_hbm.at[pl.ds(row, nr)],
                                ea0_v.at[0, pl.ds(0, nr)])
                pltpu.sync_copy(ea1_hbm.at[pl.ds(row, nr)],
                                ea1_v.at[0, pl.ds(0, nr)])
            ds = []
            for j in range(nr):
                ds.append(pltpu.async_copy(
                    xs_hbm.at[idxd_v.at[0, j]],
                    rowd_v.at[0, pl.ds(j * 128, 128)], semg0))
                ds.append(pltpu.async_copy(
                    xn_hbm.at[idxs_v.at[0, j]],
                    rows_v.at[0, pl.ds(j * 128, 128)], semg0))
            for d in ds:
                d.wait()

            def grp16(g, c2):
                a1, a2, b1, b2 = c2
                if use_ea:
                    va0 = ea0_v[0, g // 8, pl.ds((g % 8) * L, L)]
                    va1 = ea1_v[0, g // 8, pl.ds((g % 8) * L, L)]
                for u in range(L):
                    e = g * L + u
                    m = rowd_v[0, e] + rows_v[0, e]
                    if use_ea:
                        m = m + va0[u] * we0 + va1[u] * we1
                    if relu:
                        m = jnp.maximum(m, 0.0)
                    mb_v[0, e] = m
                    if u % 2 == 0:
                        a1 = a1 + m
                        a2 = a2 + m * m
                    else:
                        b1 = b1 + m
                        b2 = b2 + m * m
                return (a1, a2, b1, b2)

            carry = lax.fori_loop(0, ne // L, grp16, carry)
            pltpu.sync_copy(mb_v.at[0, pl.ds(0, ne)],
                            m_hbm.at[pl.ds(base, ne)])
            return carry

        if tail:
            carry = lax.cond(wid == NW - 1, do_tail, lambda c: c, carry)

        a1, a2, b1, b2 = carry
        st_v[0] = a1 + b1
        st_v[1] = a2 + b2
        pltpu.sync_copy(st_v, stats_hbm.at[wid])

    return pl.kernel(
        body,
        out_type=(
            jax.ShapeDtypeStruct((e_real, L), jnp.float32),
            jax.ShapeDtypeStruct((NW, 2, L), jnp.float32),
        ),
        mesh=_sc_mesh(),
        compiler_params=_SC_PARAMS,
        scratch_types=[
            pltpu.VMEM((2, chr1, 128), jnp.int32),
            pltpu.VMEM((2, chr1, 128), jnp.int32),
            pltpu.VMEM((2, chr1, 128), jnp.float32),
            pltpu.VMEM((2, chr1, 128), jnp.float32),
            pltpu.VMEM((2, nec, L), jnp.float32),
            pltpu.VMEM((2, nec, L), jnp.float32),
            pltpu.VMEM((2, nec, L), jnp.float32),
            pltpu.VMEM((2, L), jnp.float32),
            pltpu.VMEM((2, L), jnp.float32),
            pltpu.SemaphoreType.DMA,
            pltpu.SemaphoreType.DMA,
            pltpu.SemaphoreType.DMA,
            pltpu.SemaphoreType.DMA,
            pltpu.SemaphoreType.DMA,
        ],
    )


def _make_sc_pass2(e_real, n_pad):
    rows = e_real // 128
    chr2 = 8
    ch2 = chr2 * 128
    rpt = n_pad // NS
    sizes = []
    left = rpt
    while left > 0:
        sz = min(left, 512)
        sizes.append(sz)
        left -= sz

    def body(m_hbm, dst2_hbm, ss_hbm,
             out_hbm,
             acc_sp, idxd_v, yb_v, zb_v, ss_v, sem):
        cid = lax.axis_index("c")
        sid = lax.axis_index("s")

        def zrow(i, _):
            zb_v[i] = jnp.zeros((L,), jnp.float32)
            return 0

        lax.fori_loop(0, 512, zrow, 0)
        rb = sid * rpt
        off = 0
        for sz in sizes:
            pltpu.sync_copy(zb_v.at[pl.ds(0, sz)],
                            acc_sp.at[pl.ds(_al(rb + off, 8), sz)])
            off += sz
        plsc.subcore_barrier()

        pltpu.sync_copy(ss_hbm, ss_v)
        scale = ss_v[0]
        shift = ss_v[1]

        def do_block(row, nr, _):
            ne = nr * 128
            row = _al(row, 8)
            base = _al(row * 128, 1024)
            pltpu.sync_copy(dst2_hbm.at[pl.ds(row, nr)],
                            idxd_v.at[pl.ds(0, nr)])
            pltpu.sync_copy(m_hbm.at[pl.ds(base, ne)],
                            yb_v.at[pl.ds(0, ne)])

            def edge4(i, __):
                for u in range(4):
                    e = i * 4 + u
                    yb_v[e] = jnp.maximum(yb_v[e] * scale + shift, 0.0)
                return 0

            lax.fori_loop(0, ne // 4, edge4, 0)
            descs = []
            for j in range(nr):
                descs.append(pltpu.async_copy(
                    yb_v.at[pl.ds(j * 128, 128)],
                    acc_sp.at[idxd_v.at[j]], sem, add=True))
            for d in descs:
                d.wait()
            return 0

        _run_blocks(rows, chr2, do_block, 0)
        plsc.subcore_barrier()

        off = 0
        for sz in sizes:
            pltpu.sync_copy(acc_sp.at[pl.ds(_al(rb + off, 8), sz)],
                            zb_v.at[pl.ds(0, sz)])
            pltpu.sync_copy(zb_v.at[pl.ds(0, sz)],
                            out_hbm.at[cid, pl.ds(_al(rb + off, 8), sz)])
            off += sz

    return pl.kernel(
        body,
        out_type=jax.ShapeDtypeStruct((NC, n_pad, L), jnp.float32),
        mesh=_sc_mesh(),
        compiler_params=_SC_PARAMS,
        scratch_types=[
            pltpu.VMEM_SHARED((n_pad, L), jnp.float32),
            pltpu.VMEM((chr2, 128), jnp.int32),
            pltpu.VMEM((ch2, L), jnp.float32),
            pltpu.VMEM((512, L), jnp.float32),
            pltpu.VMEM((2, L), jnp.float32),
            pltpu.SemaphoreType.DMA,
        ],
    )


def _make_sc_head2(e_real):
    rows = e_real // 128

    def body(m_hbm, sw_hbm, dir_hbm, mv_v, ob_v, sw_v, sem):
        pltpu.sync_copy(sw_hbm, sw_v)
        ws2 = sw_v[0]
        c0 = sw_v[1][0]
        lanes = lax.iota(jnp.int32, L)

        def do_block(row, nr, _):
            ne = nr * 128
            base = _al(_al(row, 8) * 128, 1024)
            pltpu.sync_copy(m_hbm.at[pl.ds(base, ne)],
                            mv_v.at[pl.ds(0, ne)])

            def grp(g, __):
                acc = jnp.zeros((L,), jnp.float32)
                for i in range(L):
                    t = mv_v[g * L + i] * ws2
                    s = jnp.sum(t, axis=0) + c0
                    acc = jnp.where(lanes == i, s, acc)
                ob_v[pl.ds(g * L, L)] = 1.0 / (1.0 + jnp.exp(-1.5 * acc))
                return 0

            lax.fori_loop(0, ne // L, grp, 0)
            pltpu.sync_copy(ob_v.at[pl.ds(0, ne)],
                            dir_hbm.at[pl.ds(base, ne)])
            return 0

        _run_blocks(rows, CHR, do_block, 0)

    return pl.kernel(
        body,
        out_type=jax.ShapeDtypeStruct((e_real,), jnp.float32),
        mesh=_sc_mesh(),
        compiler_params=_SC_PARAMS,
        scratch_types=[
            pltpu.VMEM((CH, L), jnp.float32),
            pltpu.VMEM((CH,), jnp.float32),
            pltpu.VMEM((2, L), jnp.float32),
            pltpu.SemaphoreType.DMA,
        ],
    )


_BLK = 256


def _tc_combine_body(a_ref, x0_ref, s_ref, ws_ref, wn_ref,
                     t_ref, xs_ref, xn_ref):
    t = (a_ref[0] + a_ref[1]
         + jnp.dot(x0_ref[...], s_ref[...],
                   preferred_element_type=jnp.float32))
    t_ref[...] = t
    xs_ref[...] = jnp.dot(t, ws_ref[...], preferred_element_type=jnp.float32)
    xn_ref[...] = jnp.dot(t, wn_ref[...], preferred_element_type=jnp.float32)


def _tc_init_body(x0_ref, s_ref, ws_ref, wn_ref, t_ref, xs_ref, xn_ref):
    t = jnp.dot(x0_ref[...], s_ref[...], preferred_element_type=jnp.float32)
    t_ref[...] = t
    xs_ref[...] = jnp.dot(t, ws_ref[...], preferred_element_type=jnp.float32)
    xn_ref[...] = jnp.dot(t, wn_ref[...], preferred_element_type=jnp.float32)


def _wspec(r, c):
    return pl.BlockSpec((r, c), lambda b: (0, 0))


def _bspec(c=128):
    return pl.BlockSpec((_BLK, c), lambda b: (b, 0))


def _tc_combine(a, x0p, sel, wsd, wnd, np8):
    nb = np8 // _BLK
    out = jax.ShapeDtypeStruct((np8, 128), jnp.float32)
    return pl.pallas_call(
        _tc_combine_body,
        grid=(nb,),
        in_specs=[
            pl.BlockSpec((NC, _BLK, 128), lambda b: (0, b, 0)),
            _bspec(8), _wspec(8, 128), _wspec(128, 128), _wspec(128, 128),
        ],
        out_specs=[_bspec(), _bspec(), _bspec()],
        out_shape=[out, out, out],
    )(a, x0p, sel, wsd, wnd)


def _tc_init(x0p, sel, wsd, wnd, np8):
    nb = np8 // _BLK
    out = jax.ShapeDtypeStruct((np8, 128), jnp.float32)
    return pl.pallas_call(
        _tc_init_body,
        grid=(nb,),
        in_specs=[_bspec(8), _wspec(8, 128), _wspec(128, 128),
                  _wspec(128, 128)],
        out_specs=[_bspec(), _bspec(), _bspec()],
        out_shape=[out, out, out],
    )(x0p, sel, wsd, wnd)


def _tc_repack_body(i0, i1, i2, ea0, ea1,
                    o0, o1, o2, o3, o4, o5, oa0, oa1):
    o0[...] = i0[0]
    o1[...] = i0[1]
    o2[...] = i1[0]
    o3[...] = i1[1]
    o4[...] = i2[0]
    o5[...] = i2[1]
    oa0[...] = ea0[...]
    oa1[...] = ea1[...]


def _tc_repack(ei, pi, di, ea0, ea1):
    e = ei.shape[1]
    blk = 16384
    nb = pl.cdiv(e, blk)
    ispec = pl.BlockSpec((2, blk), lambda b: (0, b))
    espec = pl.BlockSpec((128, 128), lambda b: (b, 0))
    out1 = jax.ShapeDtypeStruct((e,), jnp.int32)
    oute = jax.ShapeDtypeStruct(ea0.shape, jnp.float32)
    return pl.pallas_call(
        _tc_repack_body,
        grid=(nb,),
        in_specs=[ispec, ispec, ispec, espec, espec],
        out_specs=[pl.BlockSpec((blk,), lambda b: (b,))] * 6
        + [espec, espec],
        out_shape=[out1] * 6 + [oute, oute],
    )(ei, pi, di, ea0, ea1)


def _tc_head_body(t_ref, p_ref, wa_ref, wb_ref, swa_ref, swb_ref, b1_ref,
                  at_ref, bt_ref):
    t = t_ref[...]
    p = p_ref[...]
    at_ref[...] = (jnp.dot(t, wa_ref[...], preferred_element_type=jnp.float32)
                   + jnp.dot(p, swa_ref[...],
                             preferred_element_type=jnp.float32)
                   + b1_ref[...])
    bt_ref[...] = (jnp.dot(t, wb_ref[...], preferred_element_type=jnp.float32)
                   + jnp.dot(p, swb_ref[...],
                             preferred_element_type=jnp.float32))


def _tc_head(t, powerp, wad, wbd, swa, swb, b1t, np8):
    nb = np8 // _BLK
    out = jax.ShapeDtypeStruct((np8, 128), jnp.float32)
    return pl.pallas_call(
        _tc_head_body,
        grid=(nb,),
        in_specs=[_bspec(), _bspec(8), _wspec(128, 128), _wspec(128, 128),
                  _wspec(8, 128), _wspec(8, 128), _wspec(1, 128)],
        out_specs=[_bspec(), _bspec()],
        out_shape=[out, out],
    )(t, powerp, wad, wbd, swa, swb, b1t)


def _tc_pm_body(n_real, nrow_real, t_ref, w1_ref, b1_ref, g1_ref, be1_ref,
                w2_ref, b2_ref, g2_ref, be2_ref, w3_ref, b3_ref,
                fold_ref, unf_ref, ext_ref, pw_ref, acc_ref):
    p = pl.program_id(0)
    b = pl.program_id(1)
    rows = lax.broadcasted_iota(jnp.int32, (_BLK, 1), 0) + b * _BLK
    rmask = rows < nrow_real

    @pl.when(jnp.logical_and(p == 0, b == 0))
    def _():
        acc_ref[...] = jnp.zeros_like(acc_ref)

    t = t_ref[...]
    h1 = jnp.maximum(
        jnp.dot(t, w1_ref[...], preferred_element_type=jnp.float32)
        + b1_ref[...], 0.0)

    @pl.when(p == 0)
    def _():
        hm = jnp.where(rmask, h1, 0.0)
        acc_ref[0:1, :] += jnp.sum(hm, axis=0, keepdims=True)
        acc_ref[1:2, :] += jnp.sum(hm * hm, axis=0, keepdims=True)

    @pl.when(p > 0)
    def _():
        fold = fold_ref[...]
        unf = unf_ref[...]
        mu1 = jnp.dot(acc_ref[0:1, :], fold,
                      preferred_element_type=jnp.float32) / n_real
        v1 = jnp.dot(acc_ref[1:2, :], fold,
                     preferred_element_type=jnp.float32) / n_real - mu1 * mu1
        sc1 = g1_ref[...] * jnp.dot(lax.rsqrt(v1 + EPS), unf,
                                    preferred_element_type=jnp.float32)
        mu1u = jnp.dot(mu1, unf, preferred_element_type=jnp.float32)
        h1n = (h1 - mu1u) * sc1 + be1_ref[...]
        h2 = jnp.maximum(
            jnp.dot(h1n, w2_ref[...], preferred_element_type=jnp.float32)
            + b2_ref[...], 0.0)

        @pl.when(p == 1)
        def _():
            hm2 = jnp.where(rmask, h2, 0.0)
            acc_ref[2:3, :] += jnp.sum(hm2, axis=0, keepdims=True)
            acc_ref[3:4, :] += jnp.sum(hm2 * hm2, axis=0, keepdims=True)

        @pl.when(p == 2)
        def _():
            mu2 = jnp.dot(acc_ref[2:3, :], fold,
                          preferred_element_type=jnp.float32) / n_real
            v2 = jnp.dot(acc_ref[3:4, :], fold,
                         preferred_element_type=jnp.float32) / n_real
            v2 = v2 - mu2 * mu2
            sc2 = g2_ref[...] * jnp.dot(lax.rsqrt(v2 + EPS), unf,
                                        preferred_element_type=jnp.float32)
            mu2u = jnp.dot(mu2, unf, preferred_element_type=jnp.float32)
            h2n = (h2 - mu2u) * sc2 + be2_ref[...]
            h3 = (jnp.dot(h2n, w3_ref[...], preferred_element_type=jnp.float32)
                  + b3_ref[...])
            pw_ref[...] = jnp.dot(jax.nn.sigmoid(1.5 * h3), ext_ref[...],
                                  preferred_element_type=jnp.float32)


def _tc_pm(t, p, n_real, np8):
    nb = np8 // _BLK
    return pl.pallas_call(
        functools.partial(_tc_pm_body, float(n_real), n_real // 8),
        grid=(3, nb),
        in_specs=[pl.BlockSpec((_BLK, 128), lambda ph, b: (b, 0)),
                  pl.BlockSpec((128, 128), lambda ph, b: (0, 0)),
                  pl.BlockSpec((1, 128), lambda ph, b: (0, 0)),
                  pl.BlockSpec((1, 128), lambda ph, b: (0, 0)),
                  pl.BlockSpec((1, 128), lambda ph, b: (0, 0)),
                  pl.BlockSpec((128, 128), lambda ph, b: (0, 0)),
                  pl.BlockSpec((1, 128), lambda ph, b: (0, 0)),
                  pl.BlockSpec((1, 128), lambda ph, b: (0, 0)),
                  pl.BlockSpec((1, 128), lambda ph, b: (0, 0)),
                  pl.BlockSpec((128, 128), lambda ph, b: (0, 0)),
                  pl.BlockSpec((1, 128), lambda ph, b: (0, 0)),
                  pl.BlockSpec((128, 16), lambda ph, b: (0, 0)),
                  pl.BlockSpec((16, 128), lambda ph, b: (0, 0)),
                  pl.BlockSpec((128, 8), lambda ph, b: (0, 0))],
        out_specs=pl.BlockSpec((_BLK, 8), lambda ph, b: (b, 0)),
        out_shape=jax.ShapeDtypeStruct((np8, 8), jnp.float32),
        scratch_shapes=[pltpu.VMEM((8, 128), jnp.float32)],
    )(t, *p)


def _pad16(w, row_off, col_off):
    return jnp.zeros((L, L), jnp.float32).at[
        row_off:row_off + w.shape[0], col_off:col_off + w.shape[1]].set(w)


def _blockdiag(w16):
    return jnp.kron(jnp.eye(8, dtype=jnp.float32), w16)


def _tile8(row16):
    return jnp.tile(row16.reshape(1, L), (1, 8)).reshape(1, 128)


def _bn_scale_shift(stats, g, b, e_count, lane_lo, lane_n):
    s = jnp.sum(stats, axis=0)
    mu = s[0] / e_count
    var = s[1] / e_count - mu * mu
    lanev = jnp.arange(L)
    ok = (lanev >= lane_lo) & (lanev < lane_lo + lane_n)
    g16 = jnp.zeros((L,), jnp.float32).at[lane_lo:lane_lo + lane_n].set(g)
    b16 = jnp.zeros((L,), jnp.float32).at[lane_lo:lane_lo + lane_n].set(b)
    scale = jnp.where(ok, g16 * lax.rsqrt(var + EPS), 0.0)
    shift = jnp.where(ok, b16 - mu * scale, 0.0)
    return scale, shift


def kernel(x, edge_attr, edge_index, dir_edge_index, prop_edge_index, params):
    n = x.shape[0]
    e = edge_index.shape[1]
    n_pad = pl.cdiv(n, 1024) * 1024
    np8 = n_pad // 8

    def idx2(a):
        return a.reshape(e // 128, 128)

    (ei_src, ei_dst, pi_src, pi_dst, di_src, di_dst, ea0r, ea1r) = _tc_repack(
        edge_index, prop_edge_index, dir_edge_index,
        idx2(edge_attr[:, 0]), idx2(edge_attr[:, 1]))
    ei_src, ei_dst = idx2(ei_src), idx2(ei_dst)
    pi_src, pi_dst = idx2(pi_src), idx2(pi_dst)
    di_src, di_dst = idx2(di_src), idx2(di_dst)
    x0p = jnp.pad(x[:, 0], (0, n_pad - n)).reshape(np8, 8)
    sel = jnp.zeros((8, 128), jnp.float32).at[
        jnp.arange(8), jnp.arange(8) * L].set(1.0)
    fold = jnp.kron(jnp.ones((8, 1), jnp.float32), jnp.eye(L, dtype=jnp.float32))
    unf = fold.T
    ext = jnp.kron(jnp.eye(8, dtype=jnp.float32),
                   jnp.eye(L, 1, dtype=jnp.float32))
    w_dummy = jnp.zeros((2, L), jnp.float32)

    p = params
    pass1_conv = _make_sc_pass1(e, use_ea=True, relu=False)
    pass1_inter = _make_sc_pass1(e, use_ea=False, relu=False)
    pass1_head = _make_sc_pass1(e, use_ea=False, relu=True)
    pass2 = _make_sc_pass2(e, n_pad)
    head2 = _make_sc_head2(e)

    a = None
    layers = [("c1", 1), ("d1", 3), ("c2", 5), ("d2", 7), ("c3", 9),
              ("d3", 11)]
    for name, cin in layers:
        co = cin + 1
        is_conv = name[0] == "c"
        wsd = _blockdiag(_pad16(p[name + "_Ws"], 0, 1))
        wnd = _blockdiag(_pad16(p[name + "_Wn"], 0, 1))
        if a is None:
            t_cur, xs, xn = _tc_init(x0p, sel, wsd, wnd, np8)
        else:
            ap = a.reshape(NC, np8, 128)
            t_cur, xs, xn = _tc_combine(ap, x0p, sel, wsd, wnd, np8)
        xs = xs.reshape(n_pad, L)
        xn = xn.reshape(n_pad, L)
        if is_conv:
            wep = jnp.zeros((2, L), jnp.float32).at[:, 1:1 + co].set(
                p[name + "_We"])
            m, stats = pass1_conv(xs, xn, ei_dst, ei_src, ea0r, ea1r, wep)
            dst2 = ei_dst
        else:
            m, stats = pass1_inter(xs, xn, pi_dst, pi_src, ea0r, ea1r,
                                   w_dummy)
            dst2 = pi_dst
        scale, shift = _bn_scale_shift(
            stats, p[name + "_g"], p[name + "_b"], float(e), 1, co)
        ss = jnp.stack([scale, shift])
        a = pass2(m, dst2, ss)

    idd = _blockdiag(jnp.eye(L, dtype=jnp.float32))
    t_cur, _, _ = _tc_combine(a.reshape(NC, np8, 128), x0p, sel, idd, idd, np8)

    w1d = _blockdiag(_pad16(p["pm_W1"], 0, 0))
    w2d = _blockdiag(_pad16(p["pm_W2"], 0, 0))
    w3d = _blockdiag(_pad16(p["pm_W3"], 0, 0))
    pmp = (
        w1d, _tile8(jnp.pad(p["pm_b1"], (0, 8))),
        _tile8(jnp.pad(p["pm_g1"], (0, 8))),
        _tile8(jnp.pad(p["pm_be1"], (0, 8))),
        w2d, _tile8(jnp.pad(p["pm_b2"], (0, 12))),
        _tile8(jnp.pad(p["pm_g2"], (0, 12))),
        _tile8(jnp.pad(p["pm_be2"], (0, 12))),
        w3d, _tile8(jnp.pad(p["pm_b3"], (0, 15))),
        fold, unf, ext,
    )
    power_packed = _tc_pm(t_cur, pmp, n, np8)

    fw1 = p["fe_W1"]
    wad = _blockdiag(_pad16(fw1[0:13], 0, 0))
    wbd = _blockdiag(_pad16(fw1[14:27], 0, 0))
    swa = _blockdiag(jnp.pad(fw1[13], (0, 2)).reshape(1, L))
    swb = _blockdiag(jnp.pad(fw1[27], (0, 2)).reshape(1, L))
    b1t = _tile8(jnp.pad(p["fe_b1"], (0, 2)))
    at_t, bt_t = _tc_head(t_cur, power_packed, wad, wbd, swa, swb, b1t, np8)
    at_t = at_t.reshape(n_pad, L)
    bt_t = bt_t.reshape(n_pad, L)

    mh, hstats = pass1_head(at_t, bt_t, di_dst, di_src, ea0r, ea1r, w_dummy)
    hscale, hshift = _bn_scale_shift(
        hstats, p["fe_g"], p["fe_be"], float(e), 0, 14)
    w2vec = jnp.zeros((L,), jnp.float32).at[:14].set(p["fe_W2"][:, 0])
    ws2 = hscale * w2vec
    c0 = jnp.sum(hshift * w2vec) + p["fe_b2"][0]
    sw = jnp.stack([ws2, jnp.full((L,), c0, jnp.float32)])
    direction = head2(mh, sw)

    power = power_packed.reshape(n_pad, 1)[:n]
    return power, direction.reshape(e, 1)


# speedup vs baseline: 1.1739x; 1.1739x over previous
import functools

import jax
import jax.numpy as jnp
from jax import lax
from jax.experimental import pallas as pl
from jax.experimental.pallas import tpu as pltpu
from jax.experimental.pallas import tpu_sc as plsc

L = 16
NC = 2
NS = 16
NW = NC * NS
CHR = 16
CH = CHR * 128
EPS = 1e-5

_SC_PARAMS = pltpu.CompilerParams(
    use_tc_tiling_on_sc=False, needs_layout_passes=False)


@functools.cache
def _sc_mesh():
    return plsc.VectorSubcoreMesh(
        core_axis_name="c", subcore_axis_name="s",
        num_cores=NC, num_subcores=NS)


def _al(v, m):
    return pl.multiple_of(v, m)


def _tile_chunks(rows, chunk_rows):
    nchunk = rows // chunk_rows
    tail = rows % chunk_rows
    cpw = pl.cdiv(nchunk, NW)
    return nchunk, tail, cpw


def _run_blocks(rows, chunk_rows, do_block, carry):
    wid = lax.axis_index("s") * NC + lax.axis_index("c")
    nchunk, tail, cpw = _tile_chunks(rows, chunk_rows)
    c0 = jnp.minimum(wid * cpw, nchunk)
    c1 = jnp.minimum(c0 + cpw, nchunk)

    def chunk(ch, c):
        return do_block((c0 + ch) * chunk_rows, chunk_rows, c)

    carry = lax.fori_loop(0, c1 - c0, chunk, carry)
    if tail:
        carry = lax.cond(
            wid == NW - 1,
            lambda c: do_block(jnp.int32(nchunk * chunk_rows), tail, c),
            lambda c: c, carry)
    return wid, carry


def _make_sc_pass1(e_real, use_ea, relu):
    rows = e_real // 128
    chr1 = 8
    nec = chr1 * 128
    nchunk = rows // chr1
    tail = rows % chr1
    cpw = pl.cdiv(nchunk, NW)

    def body(xs_hbm, xn_hbm, dst2_hbm, src2_hbm, ea0_hbm, ea1_hbm, w_hbm,
             m_hbm, stats_hbm,
             idxd_v, idxs_v, ea0_v, ea1_v, rowd_v, rows_v, mb_v, w_v, st_v,
             semi0, semi1, semg0, semg1, semw):
        semi = (semi0, semi1)
        semg = (semg0, semg1)
        wid = lax.axis_index("s") * NC + lax.axis_index("c")
        c0 = jnp.minimum(wid * cpw, nchunk)
        c1 = jnp.minimum(c0 + cpw, nchunk)
        nch = c1 - c0
        pltpu.sync_copy(w_hbm, w_v)
        we0 = w_v[0]
        we1 = w_v[1]
        zero = jnp.zeros((L,), jnp.float32)

        def row_of(ch):
            return _al(jnp.minimum(c0 + ch, c1 - 1) * chr1, 8)

        def load_descs(ch, b, issue):
            row = row_of(ch)
            f = pltpu.async_copy if issue else (
                lambda s, d, m: pltpu.make_async_copy(s, d, m))
            ds = [f(dst2_hbm.at[pl.ds(row, chr1)], idxd_v.at[b], semi[b]),
                  f(src2_hbm.at[pl.ds(row, chr1)], idxs_v.at[b], semi[b])]
            if use_ea:
                ds.append(f(ea0_hbm.at[pl.ds(row, chr1)], ea0_v.at[b],
                            semi[b]))
                ds.append(f(ea1_hbm.at[pl.ds(row, chr1)], ea1_v.at[b],
                            semi[b]))
            return ds

        def gather_descs(b, issue):
            f = pltpu.async_copy if issue else (
                lambda s, d, m: pltpu.make_async_copy(s, d, m))
            ds = []
            for j in range(chr1):
                ds.append(f(xs_hbm.at[idxd_v.at[b, j]],
                            rowd_v.at[b, pl.ds(j * 128, 128)], semg[b]))
                ds.append(f(xn_hbm.at[idxs_v.at[b, j]],
                            rows_v.at[b, pl.ds(j * 128, 128)], semg[b]))
            return ds

        def mw_desc(ch, b, issue):
            base = _al(row_of(ch) * 128, 1024)
            src = mb_v.at[b]
            dst = m_hbm.at[pl.ds(base, nec)]
            if issue:
                return pltpu.async_copy(src, dst, semw)
            return pltpu.make_async_copy(src, dst, semw)

        def compute(b, carry):
            def grp16(g, c2):
                a1, a2, b1, b2 = c2
                if use_ea:
                    va0 = ea0_v[b, g // 8, pl.ds((g % 8) * L, L)]
                    va1 = ea1_v[b, g // 8, pl.ds((g % 8) * L, L)]
                for u in range(L):
                    e = g * L + u
                    m = rowd_v[b, e] + rows_v[b, e]
                    if use_ea:
                        m = m + va0[u] * we0 + va1[u] * we1
                    if relu:
                        m = jnp.maximum(m, 0.0)
                    mb_v[b, e] = m
                    if u % 2 == 0:
                        a1 = a1 + m
                        a2 = a2 + m * m
                    else:
                        b1 = b1 + m
                        b2 = b2 + m * m
                return (a1, a2, b1, b2)

            return lax.fori_loop(0, nec // L, grp16, carry)

        def process(ch, b, carry):
            @pl.when(ch + 1 < nch)
            def _():
                for d in load_descs(ch + 1, 1 - b, False):
                    d.wait()
                gather_descs(1 - b, True)

            for d in gather_descs(b, False):
                d.wait()

            @pl.when(ch >= 2)
            def _():
                mw_desc(ch - 2, b, False).wait()

            carry = compute(b, carry)

            @pl.when(ch + 2 < nch)
            def _():
                load_descs(ch + 2, b, True)

            mw_desc(ch, b, True)
            return carry

        for d in load_descs(0, 0, True):
            d.wait()
        gather_descs(0, True)

        @pl.when(nch > 1)
        def _():
            load_descs(1, 1, True)

        def pair(g, carry):
            carry = process(2 * g, 0, carry)
            return process(2 * g + 1, 1, carry)

        carry = lax.fori_loop(0, nch // 2, pair, (zero, zero, zero, zero))
        carry = lax.cond(
            nch % 2 == 1,
            lambda c: process(nch - 1, 0, c),
            lambda c: c, carry)
        @pl.when(nch >= 2)
        def _():
            mw_desc(nch - 2, (nch - 2) % 2, False).wait()

        mw_desc(nch - 1, (nch - 1) % 2, False).wait()

        def do_tail(carry):
            row = _al(jnp.int32(nchunk * chr1), 8)
            nr = tail
            ne = nr * 128
            base = _al(row * 128, 1024)
            pltpu.sync_copy(dst2_hbm.at[pl.ds(row, nr)],
                            idxd_v.at[0, pl.ds(0, nr)])
            pltpu.sync_copy(src2_hbm.at[pl.ds(row, nr)],
                            idxs_v.at[0, pl.ds(0, nr)])
            if use_ea:
                pltpu.sync_copy(ea0_hbm.at[pl.ds(row, nr)],
                                ea0_v.at[0, pl.ds(0, nr)])
                pltpu.sync_copy(ea1_hbm.at[pl.ds(row, nr)],
                                ea1_v.at[0, pl.ds(0, nr)])
            ds = []
            for j in range(nr):
                ds.append(pltpu.async_copy(
                    xs_hbm.at[idxd_v.at[0, j]],
                    rowd_v.at[0, pl.ds(j * 128, 128)], semg0))
                ds.append(pltpu.async_copy(
                    xn_hbm.at[idxs_v.at[0, j]],
                    rows_v.at[0, pl.ds(j * 128, 128)], semg0))
            for d in ds:
                d.wait()

            def grp16(g, c2):
                a1, a2, b1, b2 = c2
                if use_ea:
                    va0 = ea0_v[0, g // 8, pl.ds((g % 8) * L, L)]
                    va1 = ea1_v[0, g // 8, pl.ds((g % 8) * L, L)]
                for u in range(L):
                    e = g * L + u
                    m = rowd_v[0, e] + rows_v[0, e]
                    if use_ea:
                        m = m + va0[u] * we0 + va1[u] * we1
                    if relu:
                        m = jnp.maximum(m, 0.0)
                    mb_v[0, e] = m
                    if u % 2 == 0:
                        a1 = a1 + m
                        a2 = a2 + m * m
                    else:
                        b1 = b1 + m
                        b2 = b2 + m * m
                return (a1, a2, b1, b2)

            carry = lax.fori_loop(0, ne // L, grp16, carry)
            pltpu.sync_copy(mb_v.at[0, pl.ds(0, ne)],
                            m_hbm.at[pl.ds(base, ne)])
            return carry

        if tail:
            carry = lax.cond(wid == NW - 1, do_tail, lambda c: c, carry)

        a1, a2, b1, b2 = carry
        st_v[0] = a1 + b1
        st_v[1] = a2 + b2
        pltpu.sync_copy(st_v, stats_hbm.at[wid])

    return pl.kernel(
        body,
        out_type=(
            jax.ShapeDtypeStruct((e_real, L), jnp.float32),
            jax.ShapeDtypeStruct((NW, 2, L), jnp.float32),
        ),
        mesh=_sc_mesh(),
        compiler_params=_SC_PARAMS,
        scratch_types=[
            pltpu.VMEM((2, chr1, 128), jnp.int32),
            pltpu.VMEM((2, chr1, 128), jnp.int32),
            pltpu.VMEM((2, chr1, 128), jnp.float32),
            pltpu.VMEM((2, chr1, 128), jnp.float32),
            pltpu.VMEM((2, nec, L), jnp.float32),
            pltpu.VMEM((2, nec, L), jnp.float32),
            pltpu.VMEM((2, nec, L), jnp.float32),
            pltpu.VMEM((2, L), jnp.float32),
            pltpu.VMEM((2, L), jnp.float32),
            pltpu.SemaphoreType.DMA,
            pltpu.SemaphoreType.DMA,
            pltpu.SemaphoreType.DMA,
            pltpu.SemaphoreType.DMA,
            pltpu.SemaphoreType.DMA,
        ],
    )


def _make_sc_pass2(e_real, n_pad):
    rows = e_real // 128
    chr2 = 4
    ch2 = chr2 * 128
    nchunk = rows // chr2
    assert rows % chr2 == 0
    cpw = pl.cdiv(nchunk, NW)
    rpt = n_pad // NS
    sizes = []
    left = rpt
    while left > 0:
        sz = min(left, 256)
        sizes.append(sz)
        left -= sz

    def body(m_hbm, dst2_hbm, ss_hbm,
             out_hbm,
             acc_sp, idxd_v, yb_v, zb_v, ss_v, semi0, semi1, semg0, semg1):
        semi = (semi0, semi1)
        semg = (semg0, semg1)
        cid = lax.axis_index("c")
        sid = lax.axis_index("s")
        wid = sid * NC + cid
        c0 = jnp.minimum(wid * cpw, nchunk)
        c1 = jnp.minimum(c0 + cpw, nchunk)
        nch = c1 - c0

        def zrow(i, _):
            zb_v[i] = jnp.zeros((L,), jnp.float32)
            return 0

        lax.fori_loop(0, 256, zrow, 0)
        rb = sid * rpt
        off = 0
        for sz in sizes:
            pltpu.sync_copy(zb_v.at[pl.ds(0, sz)],
                            acc_sp.at[pl.ds(_al(rb + off, 8), sz)])
            off += sz
        plsc.subcore_barrier()

        pltpu.sync_copy(ss_hbm, ss_v)
        scale = ss_v[0]
        shift = ss_v[1]

        def row_of(ch):
            return _al(jnp.minimum(c0 + ch, c1 - 1) * chr2, 4)

        def load_descs(ch, b, issue):
            row = row_of(ch)
            base = _al(row * 128, 512)
            f = pltpu.async_copy if issue else (
                lambda s, d, m: pltpu.make_async_copy(s, d, m))
            return [f(dst2_hbm.at[pl.ds(row, chr2)], idxd_v.at[b], semi[b]),
                    f(m_hbm.at[pl.ds(base, ch2)], yb_v.at[b], semi[b])]

        def scat_descs(b, issue):
            f = (lambda s, d, m: pltpu.async_copy(s, d, m, add=True)) \
                if issue else (lambda s, d, m: pltpu.make_async_copy(s, d, m))
            return [f(yb_v.at[b, pl.ds(j * 128, 128)],
                      acc_sp.at[idxd_v.at[b, j]], semg[b])
                    for j in range(chr2)]

        def process(ch, b, _):
            @pl.when(ch >= 1)
            def _():
                for d in scat_descs(1 - b, False):
                    d.wait()

            @pl.when(ch + 1 < nch)
            def _():
                load_descs(ch + 1, 1 - b, True)

            for d in load_descs(ch, b, False):
                d.wait()

            def edge4(i, __):
                for u in range(4):
                    e = i * 4 + u
                    yb_v[b, e] = jnp.maximum(
                        yb_v[b, e] * scale + shift, 0.0)
                return 0

            lax.fori_loop(0, ch2 // 4, edge4, 0)
            scat_descs(b, True)
            return 0

        load_descs(0, 0, True)

        def pair(g, c):
            process(2 * g, 0, c)
            return process(2 * g + 1, 1, c)

        lax.fori_loop(0, nch // 2, pair, 0)
        lax.cond(nch % 2 == 1,
                 lambda c: process(nch - 1, 0, c),
                 lambda c: c, 0)
        lax.cond(nch % 2 == 1,
                 lambda c: [d.wait() for d in scat_descs(0, False)] and 0,
                 lambda c: [d.wait() for d in scat_descs(1, False)] and 0,
                 0)

        plsc.subcore_barrier()

        off = 0
        for sz in sizes:
            pltpu.sync_copy(acc_sp.at[pl.ds(_al(rb + off, 8), sz)],
                            zb_v.at[pl.ds(0, sz)])
            pltpu.sync_copy(zb_v.at[pl.ds(0, sz)],
                            out_hbm.at[cid, pl.ds(_al(rb + off, 8), sz)])
            off += sz

    return pl.kernel(
        body,
        out_type=jax.ShapeDtypeStruct((NC, n_pad, L), jnp.float32),
        mesh=_sc_mesh(),
        compiler_params=_SC_PARAMS,
        scratch_types=[
            pltpu.VMEM_SHARED((n_pad, L), jnp.float32),
            pltpu.VMEM((2, chr2, 128), jnp.int32),
            pltpu.VMEM((2, ch2, L), jnp.float32),
            pltpu.VMEM((256, L), jnp.float32),
            pltpu.VMEM((2, L), jnp.float32),
            pltpu.SemaphoreType.DMA,
            pltpu.SemaphoreType.DMA,
            pltpu.SemaphoreType.DMA,
            pltpu.SemaphoreType.DMA,
        ],
    )


def _make_sc_head2(e_real):
    rows = e_real // 128

    def body(m_hbm, sw_hbm, dir_hbm, mv_v, ob_v, sw_v, sem):
        pltpu.sync_copy(sw_hbm, sw_v)
        ws2 = sw_v[0]
        c0 = sw_v[1][0]
        lanes = lax.iota(jnp.int32, L)

        def do_block(row, nr, _):
            ne = nr * 128
            base = _al(_al(row, 8) * 128, 1024)
            pltpu.sync_copy(m_hbm.at[pl.ds(base, ne)],
                            mv_v.at[pl.ds(0, ne)])

            def grp(g, __):
                acc = jnp.zeros((L,), jnp.float32)
                for i in range(L):
                    t = mv_v[g * L + i] * ws2
                    s = jnp.sum(t, axis=0) + c0
                    acc = jnp.where(lanes == i, s, acc)
                ob_v[pl.ds(g * L, L)] = 1.0 / (1.0 + jnp.exp(-1.5 * acc))
                return 0

            lax.fori_loop(0, ne // L, grp, 0)
            pltpu.sync_copy(ob_v.at[pl.ds(0, ne)],
                            dir_hbm.at[pl.ds(base, ne)])
            return 0

        _run_blocks(rows, CHR, do_block, 0)

    return pl.kernel(
        body,
        out_type=jax.ShapeDtypeStruct((e_real,), jnp.float32),
        mesh=_sc_mesh(),
        compiler_params=_SC_PARAMS,
        scratch_types=[
            pltpu.VMEM((CH, L), jnp.float32),
            pltpu.VMEM((CH,), jnp.float32),
            pltpu.VMEM((2, L), jnp.float32),
            pltpu.SemaphoreType.DMA,
        ],
    )


_BLK = 256


def _tc_combine_body(a_ref, x0_ref, s_ref, ws_ref, wn_ref,
                     t_ref, xs_ref, xn_ref):
    t = (a_ref[0] + a_ref[1]
         + jnp.dot(x0_ref[...], s_ref[...],
                   preferred_element_type=jnp.float32))
    t_ref[...] = t
    xs_ref[...] = jnp.dot(t, ws_ref[...], preferred_element_type=jnp.float32)
    xn_ref[...] = jnp.dot(t, wn_ref[...], preferred_element_type=jnp.float32)


def _tc_init_body(x0_ref, s_ref, ws_ref, wn_ref, t_ref, xs_ref, xn_ref):
    t = jnp.dot(x0_ref[...], s_ref[...], preferred_element_type=jnp.float32)
    t_ref[...] = t
    xs_ref[...] = jnp.dot(t, ws_ref[...], preferred_element_type=jnp.float32)
    xn_ref[...] = jnp.dot(t, wn_ref[...], preferred_element_type=jnp.float32)


def _wspec(r, c):
    return pl.BlockSpec((r, c), lambda b: (0, 0))


def _bspec(c=128):
    return pl.BlockSpec((_BLK, c), lambda b: (b, 0))


def _tc_combine(a, x0p, sel, wsd, wnd, np8):
    nb = np8 // _BLK
    out = jax.ShapeDtypeStruct((np8, 128), jnp.float32)
    return pl.pallas_call(
        _tc_combine_body,
        grid=(nb,),
        in_specs=[
            pl.BlockSpec((NC, _BLK, 128), lambda b: (0, b, 0)),
            _bspec(8), _wspec(8, 128), _wspec(128, 128), _wspec(128, 128),
        ],
        out_specs=[_bspec(), _bspec(), _bspec()],
        out_shape=[out, out, out],
    )(a, x0p, sel, wsd, wnd)


def _tc_init(x0p, sel, wsd, wnd, np8):
    nb = np8 // _BLK
    out = jax.ShapeDtypeStruct((np8, 128), jnp.float32)
    return pl.pallas_call(
        _tc_init_body,
        grid=(nb,),
        in_specs=[_bspec(8), _wspec(8, 128), _wspec(128, 128),
                  _wspec(128, 128)],
        out_specs=[_bspec(), _bspec(), _bspec()],
        out_shape=[out, out, out],
    )(x0p, sel, wsd, wnd)


def _tc_repack_body(i0, i1, i2, ea0, ea1,
                    o0, o1, o2, o3, o4, o5, oa0, oa1):
    o0[...] = i0[0]
    o1[...] = i0[1]
    o2[...] = i1[0]
    o3[...] = i1[1]
    o4[...] = i2[0]
    o5[...] = i2[1]
    oa0[...] = ea0[...]
    oa1[...] = ea1[...]


def _tc_repack(ei, pi, di, ea0, ea1):
    e = ei.shape[1]
    blk = 16384
    nb = pl.cdiv(e, blk)
    ispec = pl.BlockSpec((2, blk), lambda b: (0, b))
    espec = pl.BlockSpec((128, 128), lambda b: (b, 0))
    out1 = jax.ShapeDtypeStruct((e,), jnp.int32)
    oute = jax.ShapeDtypeStruct(ea0.shape, jnp.float32)
    return pl.pallas_call(
        _tc_repack_body,
        grid=(nb,),
        in_specs=[ispec, ispec, ispec, espec, espec],
        out_specs=[pl.BlockSpec((blk,), lambda b: (b,))] * 6
        + [espec, espec],
        out_shape=[out1] * 6 + [oute, oute],
    )(ei, pi, di, ea0, ea1)


def _tc_head_body(t_ref, p_ref, wa_ref, wb_ref, swa_ref, swb_ref, b1_ref,
                  at_ref, bt_ref):
    t = t_ref[...]
    p = p_ref[...]
    at_ref[...] = (jnp.dot(t, wa_ref[...], preferred_element_type=jnp.float32)
                   + jnp.dot(p, swa_ref[...],
                             preferred_element_type=jnp.float32)
                   + b1_ref[...])
    bt_ref[...] = (jnp.dot(t, wb_ref[...], preferred_element_type=jnp.float32)
                   + jnp.dot(p, swb_ref[...],
                             preferred_element_type=jnp.float32))


def _tc_head(t, powerp, wad, wbd, swa, swb, b1t, np8):
    nb = np8 // _BLK
    out = jax.ShapeDtypeStruct((np8, 128), jnp.float32)
    return pl.pallas_call(
        _tc_head_body,
        grid=(nb,),
        in_specs=[_bspec(), _bspec(8), _wspec(128, 128), _wspec(128, 128),
                  _wspec(8, 128), _wspec(8, 128), _wspec(1, 128)],
        out_specs=[_bspec(), _bspec()],
        out_shape=[out, out],
    )(t, powerp, wad, wbd, swa, swb, b1t)


def _tc_pm_body(n_real, nrow_real, t_ref, w1_ref, b1_ref, g1_ref, be1_ref,
                w2_ref, b2_ref, g2_ref, be2_ref, w3_ref, b3_ref,
                fold_ref, unf_ref, ext_ref, pw_ref, acc_ref):
    p = pl.program_id(0)
    b = pl.program_id(1)
    rows = lax.broadcasted_iota(jnp.int32, (_BLK, 1), 0) + b * _BLK
    rmask = rows < nrow_real

    @pl.when(jnp.logical_and(p == 0, b == 0))
    def _():
        acc_ref[...] = jnp.zeros_like(acc_ref)

    t = t_ref[...]
    h1 = jnp.maximum(
        jnp.dot(t, w1_ref[...], preferred_element_type=jnp.float32)
        + b1_ref[...], 0.0)

    @pl.when(p == 0)
    def _():
        hm = jnp.where(rmask, h1, 0.0)
        acc_ref[0:1, :] += jnp.sum(hm, axis=0, keepdims=True)
        acc_ref[1:2, :] += jnp.sum(hm * hm, axis=0, keepdims=True)

    @pl.when(p > 0)
    def _():
        fold = fold_ref[...]
        unf = unf_ref[...]
        mu1 = jnp.dot(acc_ref[0:1, :], fold,
                      preferred_element_type=jnp.float32) / n_real
        v1 = jnp.dot(acc_ref[1:2, :], fold,
                     preferred_element_type=jnp.float32) / n_real - mu1 * mu1
        sc1 = g1_ref[...] * jnp.dot(lax.rsqrt(v1 + EPS), unf,
                                    preferred_element_type=jnp.float32)
        mu1u = jnp.dot(mu1, unf, preferred_element_type=jnp.float32)
        h1n = (h1 - mu1u) * sc1 + be1_ref[...]
        h2 = jnp.maximum(
            jnp.dot(h1n, w2_ref[...], preferred_element_type=jnp.float32)
            + b2_ref[...], 0.0)

        @pl.when(p == 1)
        def _():
            hm2 = jnp.where(rmask, h2, 0.0)
            acc_ref[2:3, :] += jnp.sum(hm2, axis=0, keepdims=True)
            acc_ref[3:4, :] += jnp.sum(hm2 * hm2, axis=0, keepdims=True)

        @pl.when(p == 2)
        def _():
            mu2 = jnp.dot(acc_ref[2:3, :], fold,
                          preferred_element_type=jnp.float32) / n_real
            v2 = jnp.dot(acc_ref[3:4, :], fold,
                         preferred_element_type=jnp.float32) / n_real
            v2 = v2 - mu2 * mu2
            sc2 = g2_ref[...] * jnp.dot(lax.rsqrt(v2 + EPS), unf,
                                        preferred_element_type=jnp.float32)
            mu2u = jnp.dot(mu2, unf, preferred_element_type=jnp.float32)
            h2n = (h2 - mu2u) * sc2 + be2_ref[...]
            h3 = (jnp.dot(h2n, w3_ref[...], preferred_element_type=jnp.float32)
                  + b3_ref[...])
            pw_ref[...] = jnp.dot(jax.nn.sigmoid(1.5 * h3), ext_ref[...],
                                  preferred_element_type=jnp.float32)


def _tc_pm(t, p, n_real, np8):
    nb = np8 // _BLK
    return pl.pallas_call(
        functools.partial(_tc_pm_body, float(n_real), n_real // 8),
        grid=(3, nb),
        in_specs=[pl.BlockSpec((_BLK, 128), lambda ph, b: (b, 0)),
                  pl.BlockSpec((128, 128), lambda ph, b: (0, 0)),
                  pl.BlockSpec((1, 128), lambda ph, b: (0, 0)),
                  pl.BlockSpec((1, 128), lambda ph, b: (0, 0)),
                  pl.BlockSpec((1, 128), lambda ph, b: (0, 0)),
                  pl.BlockSpec((128, 128), lambda ph, b: (0, 0)),
                  pl.BlockSpec((1, 128), lambda ph, b: (0, 0)),
                  pl.BlockSpec((1, 128), lambda ph, b: (0, 0)),
                  pl.BlockSpec((1, 128), lambda ph, b: (0, 0)),
                  pl.BlockSpec((128, 128), lambda ph, b: (0, 0)),
                  pl.BlockSpec((1, 128), lambda ph, b: (0, 0)),
                  pl.BlockSpec((128, 16), lambda ph, b: (0, 0)),
                  pl.BlockSpec((16, 128), lambda ph, b: (0, 0)),
                  pl.BlockSpec((128, 8), lambda ph, b: (0, 0))],
        out_specs=pl.BlockSpec((_BLK, 8), lambda ph, b: (b, 0)),
        out_shape=jax.ShapeDtypeStruct((np8, 8), jnp.float32),
        scratch_shapes=[pltpu.VMEM((8, 128), jnp.float32)],
    )(t, *p)


def _pad16(w, row_off, col_off):
    return jnp.zeros((L, L), jnp.float32).at[
        row_off:row_off + w.shape[0], col_off:col_off + w.shape[1]].set(w)


def _blockdiag(w16):
    return jnp.kron(jnp.eye(8, dtype=jnp.float32), w16)


def _tile8(row16):
    return jnp.tile(row16.reshape(1, L), (1, 8)).reshape(1, 128)


def _bn_scale_shift(stats, g, b, e_count, lane_lo, lane_n):
    s = jnp.sum(stats, axis=0)
    mu = s[0] / e_count
    var = s[1] / e_count - mu * mu
    lanev = jnp.arange(L)
    ok = (lanev >= lane_lo) & (lanev < lane_lo + lane_n)
    g16 = jnp.zeros((L,), jnp.float32).at[lane_lo:lane_lo + lane_n].set(g)
    b16 = jnp.zeros((L,), jnp.float32).at[lane_lo:lane_lo + lane_n].set(b)
    scale = jnp.where(ok, g16 * lax.rsqrt(var + EPS), 0.0)
    shift = jnp.where(ok, b16 - mu * scale, 0.0)
    return scale, shift


def kernel(x, edge_attr, edge_index, dir_edge_index, prop_edge_index, params):
    n = x.shape[0]
    e = edge_index.shape[1]
    n_pad = pl.cdiv(n, 1024) * 1024
    np8 = n_pad // 8

    def idx2(a):
        return a.reshape(e // 128, 128)

    (ei_src, ei_dst, pi_src, pi_dst, di_src, di_dst, ea0r, ea1r) = _tc_repack(
        edge_index, prop_edge_index, dir_edge_index,
        idx2(edge_attr[:, 0]), idx2(edge_attr[:, 1]))
    ei_src, ei_dst = idx2(ei_src), idx2(ei_dst)
    pi_src, pi_dst = idx2(pi_src), idx2(pi_dst)
    di_src, di_dst = idx2(di_src), idx2(di_dst)
    x0p = jnp.pad(x[:, 0], (0, n_pad - n)).reshape(np8, 8)
    sel = jnp.zeros((8, 128), jnp.float32).at[
        jnp.arange(8), jnp.arange(8) * L].set(1.0)
    fold = jnp.kron(jnp.ones((8, 1), jnp.float32), jnp.eye(L, dtype=jnp.float32))
    unf = fold.T
    ext = jnp.kron(jnp.eye(8, dtype=jnp.float32),
                   jnp.eye(L, 1, dtype=jnp.float32))
    w_dummy = jnp.zeros((2, L), jnp.float32)

    p = params
    pass1_conv = _make_sc_pass1(e, use_ea=True, relu=False)
    pass1_inter = _make_sc_pass1(e, use_ea=False, relu=False)
    pass1_head = _make_sc_pass1(e, use_ea=False, relu=True)
    pass2 = _make_sc_pass2(e, n_pad)
    head2 = _make_sc_head2(e)

    a = None
    layers = [("c1", 1), ("d1", 3), ("c2", 5), ("d2", 7), ("c3", 9),
              ("d3", 11)]
    for name, cin in layers:
        co = cin + 1
        is_conv = name[0] == "c"
        wsd = _blockdiag(_pad16(p[name + "_Ws"], 0, 1))
        wnd = _blockdiag(_pad16(p[name + "_Wn"], 0, 1))
        if a is None:
            t_cur, xs, xn = _tc_init(x0p, sel, wsd, wnd, np8)
        else:
            ap = a.reshape(NC, np8, 128)
            t_cur, xs, xn = _tc_combine(ap, x0p, sel, wsd, wnd, np8)
        xs = xs.reshape(n_pad, L)
        xn = xn.reshape(n_pad, L)
        if is_conv:
            wep = jnp.zeros((2, L), jnp.float32).at[:, 1:1 + co].set(
                p[name + "_We"])
            m, stats = pass1_conv(xs, xn, ei_dst, ei_src, ea0r, ea1r, wep)
            dst2 = ei_dst
        else:
            m, stats = pass1_inter(xs, xn, pi_dst, pi_src, ea0r, ea1r,
                                   w_dummy)
            dst2 = pi_dst
        scale, shift = _bn_scale_shift(
            stats, p[name + "_g"], p[name + "_b"], float(e), 1, co)
        ss = jnp.stack([scale, shift])
        a = pass2(m, dst2, ss)

    idd = _blockdiag(jnp.eye(L, dtype=jnp.float32))
    t_cur, _, _ = _tc_combine(a.reshape(NC, np8, 128), x0p, sel, idd, idd, np8)

    w1d = _blockdiag(_pad16(p["pm_W1"], 0, 0))
    w2d = _blockdiag(_pad16(p["pm_W2"], 0, 0))
    w3d = _blockdiag(_pad16(p["pm_W3"], 0, 0))
    pmp = (
        w1d, _tile8(jnp.pad(p["pm_b1"], (0, 8))),
        _tile8(jnp.pad(p["pm_g1"], (0, 8))),
        _tile8(jnp.pad(p["pm_be1"], (0, 8))),
        w2d, _tile8(jnp.pad(p["pm_b2"], (0, 12))),
        _tile8(jnp.pad(p["pm_g2"], (0, 12))),
        _tile8(jnp.pad(p["pm_be2"], (0, 12))),
        w3d, _tile8(jnp.pad(p["pm_b3"], (0, 15))),
        fold, unf, ext,
    )
    power_packed = _tc_pm(t_cur, pmp, n, np8)

    fw1 = p["fe_W1"]
    wad = _blockdiag(_pad16(fw1[0:13], 0, 0))
    wbd = _blockdiag(_pad16(fw1[14:27], 0, 0))
    swa = _blockdiag(jnp.pad(fw1[13], (0, 2)).reshape(1, L))
    swb = _blockdiag(jnp.pad(fw1[27], (0, 2)).reshape(1, L))
    b1t = _tile8(jnp.pad(p["fe_b1"], (0, 2)))
    at_t, bt_t = _tc_head(t_cur, power_packed, wad, wbd, swa, swb, b1t, np8)
    at_t = at_t.reshape(n_pad, L)
    bt_t = bt_t.reshape(n_pad, L)

    mh, hstats = pass1_head(at_t, bt_t, di_dst, di_src, ea0r, ea1r, w_dummy)
    hscale, hshift = _bn_scale_shift(
        hstats, p["fe_g"], p["fe_be"], float(e), 0, 14)
    w2vec = jnp.zeros((L,), jnp.float32).at[:14].set(p["fe_W2"][:, 0])
    ws2 = hscale * w2vec
    c0 = jnp.sum(hshift * w2vec) + p["fe_b2"][0]
    sw = jnp.stack([ws2, jnp.full((L,), c0, jnp.float32)])
    direction = head2(mh, sw)

    power = power_packed.reshape(n_pad, 1)[:n]
    return power, direction.reshape(e, 1)
